# baseline (speedup 1.0000x reference)
import functools
import jax
import jax.numpy as jnp
from jax import lax
from jax.experimental import pallas as pl
from jax.experimental.pallas import tpu as pltpu
from jax.experimental.pallas import tpu_sc as plsc

_NC = 2
_NS = 16
_NW = _NC * _NS
_CHUNK = 128
_GG = 5
_LANES = 128


def _gather_body(nch, d_e, idx_i_hbm, idx_c_hbm, idx_b_hbm, price_hbm,
                 zeros_hbm, t_item, t_cate, t_brand, out_p,
                 idx_v, pr_v, rows_i, rows_c, rows_b, tail, gsem, wsem):
  wid = lax.axis_index("s") * _NC + lax.axis_index("c")
  base = wid * nch * _CHUNK
  rows_g = _GG * _CHUNK
  pltpu.sync_copy(idx_i_hbm.at[wid], idx_v.at[0])
  pltpu.sync_copy(idx_c_hbm.at[wid], idx_v.at[1])
  pltpu.sync_copy(idx_b_hbm.at[wid], idx_v.at[2])
  pltpu.sync_copy(price_hbm.at[wid], pr_v)
  pltpu.sync_copy(zeros_hbm, tail)

  tables = ((t_item, rows_i, 0), (t_cate, rows_c, 1), (t_brand, rows_b, 2))
  lane0 = jnp.zeros((16,), jnp.int32)
  iota16 = lax.iota(jnp.int32, 16)

  def group_body(g, _):
    row0 = base + g * rows_g
    cps = []
    for tbl, rows, t in tables:
      for j in range(_GG):
        cps.append(pltpu.async_copy(
            tbl.at[idx_v.at[t, g * _GG + j]],
            rows.at[pl.ds(j * _CHUNK, _CHUNK)], gsem))
    for j in range(_GG):
      c = g * _GG + j
      for k in range(_CHUNK // 16):
        pv = pr_v[c, pl.ds(16 * k, 16)]
        rows_idx = iota16 + (j * _CHUNK + 16 * k)
        plsc.store_scatter(tail, [rows_idx, lane0], pv)
    for cp in cps:
      cp.wait()
    wr = [pltpu.async_copy(
        rows, out_p.at[pl.ds(row0, rows_g), pl.ds(t * d_e, d_e)], wsem)
        for _tbl, rows, t in tables]
    wr.append(pltpu.async_copy(
        tail, out_p.at[pl.ds(row0, rows_g), pl.ds(3 * d_e, d_e)], wsem))
    for cp in wr:
      cp.wait()
    return 0

  lax.fori_loop(0, nch // _GG, group_body, 0)


def _sc_gather(idx_i, idx_c, idx_b, price, zeros, emb_item, emb_cate,
               emb_brand):
  nw, nch, chunk = idx_i.shape
  n = nw * nch * chunk
  d_e = emb_item.shape[1]
  mesh = plsc.VectorSubcoreMesh(core_axis_name="c", subcore_axis_name="s",
                                num_cores=_NC, num_subcores=_NS)
  body = functools.partial(_gather_body, nch, d_e)
  return pl.kernel(
      body,
      out_type=jax.ShapeDtypeStruct((n, _LANES), jnp.float32),
      mesh=mesh,
      scratch_types=[
          pltpu.VMEM((3, nch, chunk), jnp.int32),
          pltpu.VMEM((nch, chunk), jnp.float32),
          pltpu.VMEM((_GG * chunk, d_e), jnp.float32),
          pltpu.VMEM((_GG * chunk, d_e), jnp.float32),
          pltpu.VMEM((_GG * chunk, d_e), jnp.float32),
          pltpu.VMEM((_GG * chunk, d_e), jnp.float32),
          pltpu.SemaphoreType.DMA,
          pltpu.SemaphoreType.DMA,
      ],
      compiler_params=pltpu.CompilerParams(use_tc_tiling_on_sc=False,
                                           needs_layout_passes=False),
  )(idx_i, idx_c, idx_b, price, zeros, emb_item, emb_cate, emb_brand)


def _repack_body(t_ref, iden_ref, out_ref):
  iden = iden_ref[...]
  pieces = [
      lax.dot_general(iden, t_ref[:, a * _LANES:(a + 1) * _LANES],
                      (((1,), (1,)), ((), ())),
                      preferred_element_type=jnp.float32)
      for a in range(4)
  ]
  out_ref[...] = jnp.concatenate(pieces, axis=1)


def _repack_table(emb, blk=512):
  v, d = emb.shape
  tt = emb.T
  g = -(-v // blk)
  rows = blk * d // _LANES
  out = pl.pallas_call(
      _repack_body,
      grid=(g,),
      in_specs=[
          pl.BlockSpec((d, blk), lambda i: (0, i)),
          pl.BlockSpec((_LANES, _LANES), lambda i: (0, 0)),
      ],
      out_specs=pl.BlockSpec((rows, _LANES), lambda i: (i, 0)),
      out_shape=jax.ShapeDtypeStruct((g * rows, _LANES), jnp.float32),
  )(tt, jnp.eye(_LANES, dtype=jnp.float32))
  return out.reshape(g * blk, d)


def _remap_idx(v):
  return (v & ~511) + ((v & 127) << 2) + ((v >> 7) & 3)


def _mlp_body(p_ref, mm_ref, wp, wm, b1e, w2, b2, out_ref):
  h = (jnp.dot(p_ref[...], wp[...], preferred_element_type=jnp.float32)
       + jnp.dot(mm_ref[...], wm[...], preferred_element_type=jnp.float32)
       + b1e[...])
  h = jnp.maximum(h, 0.0)
  out_ref[...] = jnp.dot(h, w2[...], preferred_element_type=jnp.float32) + b2[...]


def _tc_mlp(p, mm2, wp, wm, b1e, w2, b2, bl=512):
  n, d_mm = mm2.shape
  d_dnn = wp.shape[1]
  d_hid = w2.shape[1]
  grid = (n // bl,)
  full = lambda r, c: pl.BlockSpec((r, c), lambda i: (0, 0))
  return pl.pallas_call(
      _mlp_body,
      grid=grid,
      in_specs=[
          pl.BlockSpec((bl, _LANES), lambda i: (i, 0)),
          pl.BlockSpec((bl, d_mm), lambda i: (i, 0)),
          full(_LANES, d_dnn),
          full(d_mm, d_dnn),
          full(1, d_dnn),
          full(d_dnn, d_hid),
          full(1, d_hid),
      ],
      out_specs=pl.BlockSpec((bl, d_hid), lambda i: (i, 0)),
      out_shape=jax.ShapeDtypeStruct((n, d_hid), jnp.float32),
  )(p, mm2, wp, wm, b1e, w2, b2)


def kernel(seq_id, item_mask, feat_cate, feat_brand, feat_price, feat_mm,
           emb_item, emb_cate, emb_brand, mm_W, mm_b, W1, b1, W2, b2):
  b, l = seq_id.shape
  n = b * l
  d_e = emb_item.shape[1]
  d_mm_in, d_mm = mm_W.shape
  d_dnn = W1.shape[1]
  assert n % (_NW * _CHUNK) == 0
  nch = n // (_NW * _CHUNK)

  ids = _remap_idx(
      (seq_id * item_mask).astype(jnp.int32)).T.reshape(_NW, nch, _CHUNK)
  idx_c = _remap_idx(
      feat_cate.astype(jnp.int32)).T.reshape(_NW, nch, _CHUNK)
  idx_b = _remap_idx(
      feat_brand.astype(jnp.int32)).T.reshape(_NW, nch, _CHUNK)
  price = feat_price.astype(jnp.float32).T.reshape(_NW, nch, _CHUNK)
  zeros = jnp.zeros((_GG * _CHUNK, d_e), jnp.float32)

  p = _sc_gather(ids, idx_c, idx_b, price, zeros,
                 _repack_table(emb_item), _repack_table(emb_cate),
                 _repack_table(emb_brand))

  n_emb = 3 * d_e + 1
  w1_mm = W1[n_emb:]
  wp = jnp.concatenate(
      [W1[:n_emb], jnp.zeros((_LANES - n_emb, d_dnn), W1.dtype)], axis=0)
  wm = mm_W @ w1_mm
  b1e = (b1 + mm_b @ w1_mm)[None, :]

  mm2 = jnp.transpose(feat_mm, (1, 0, 2)).reshape(n, d_mm_in)
  out = _tc_mlp(p, mm2, wp, wm, b1e, W2, b2[None, :])
  return jnp.transpose(out.reshape(l, b, out.shape[-1]), (1, 0, 2))


# speedup vs baseline: 1.2613x; 1.2613x over previous
import functools
import jax
import jax.numpy as jnp
from jax import lax
from jax.experimental import pallas as pl
from jax.experimental.pallas import tpu as pltpu
from jax.experimental.pallas import tpu_sc as plsc

_NC = 2
_NS = 16
_NW = _NC * _NS
_CHUNK = 128
_GG = 5
_LANES = 128


def _gather_body(nch, d_e, idx_i_hbm, idx_c_hbm, idx_b_hbm, price_hbm,
                 zeros_hbm, t_item, t_cate, t_brand, out_p,
                 idx_v, pr_v, rows_i, rows_c, rows_b, tail, gsem, wsem):
  wid = lax.axis_index("s") * _NC + lax.axis_index("c")
  base = wid * nch * _CHUNK
  rows_g = _GG * _CHUNK
  pltpu.sync_copy(idx_i_hbm.at[wid], idx_v.at[0])
  pltpu.sync_copy(idx_c_hbm.at[wid], idx_v.at[1])
  pltpu.sync_copy(idx_b_hbm.at[wid], idx_v.at[2])
  pltpu.sync_copy(price_hbm.at[wid], pr_v)
  pltpu.sync_copy(zeros_hbm, tail)

  tables = ((t_item, rows_i, 0), (t_cate, rows_c, 1), (t_brand, rows_b, 2))
  lane0 = jnp.zeros((16,), jnp.int32)
  iota16 = lax.iota(jnp.int32, 16)

  def group_body(g, _):
    row0 = base + g * rows_g
    cps = []
    for tbl, rows, t in tables:
      for j in range(_GG):
        cps.append(pltpu.async_copy(
            tbl.at[idx_v.at[t, g * _GG + j]],
            rows.at[pl.ds(j * _CHUNK, _CHUNK)], gsem))
    for j in range(_GG):
      c = g * _GG + j
      for k in range(_CHUNK // 16):
        pv = pr_v[c, pl.ds(16 * k, 16)]
        rows_idx = iota16 + (j * _CHUNK + 16 * k)
        plsc.store_scatter(tail, [rows_idx, lane0], pv)
    for cp in cps:
      cp.wait()
    wr = [pltpu.async_copy(
        rows, out_p.at[pl.ds(row0, rows_g), pl.ds(t * d_e, d_e)], wsem)
        for _tbl, rows, t in tables]
    wr.append(pltpu.async_copy(
        tail, out_p.at[pl.ds(row0, rows_g), pl.ds(3 * d_e, d_e)], wsem))
    for cp in wr:
      cp.wait()
    return 0

  lax.fori_loop(0, nch // _GG, group_body, 0)


def _sc_gather(idx_i, idx_c, idx_b, price, zeros, emb_item, emb_cate,
               emb_brand):
  nw, nch, chunk = idx_i.shape
  n = nw * nch * chunk
  d_e = emb_item.shape[1]
  mesh = plsc.VectorSubcoreMesh(core_axis_name="c", subcore_axis_name="s",
                                num_cores=_NC, num_subcores=_NS)
  body = functools.partial(_gather_body, nch, d_e)
  return pl.kernel(
      body,
      out_type=jax.ShapeDtypeStruct((n, _LANES), jnp.float32),
      mesh=mesh,
      scratch_types=[
          pltpu.VMEM((3, nch, chunk), jnp.int32),
          pltpu.VMEM((nch, chunk), jnp.float32),
          pltpu.VMEM((_GG * chunk, d_e), jnp.float32),
          pltpu.VMEM((_GG * chunk, d_e), jnp.float32),
          pltpu.VMEM((_GG * chunk, d_e), jnp.float32),
          pltpu.VMEM((_GG * chunk, d_e), jnp.float32),
          pltpu.SemaphoreType.DMA,
          pltpu.SemaphoreType.DMA,
      ],
      compiler_params=pltpu.CompilerParams(use_tc_tiling_on_sc=False,
                                           needs_layout_passes=False),
  )(idx_i, idx_c, idx_b, price, zeros, emb_item, emb_cate, emb_brand)


_RPK_U = 8


def _repack_body(t_ref, iden_ref, out_ref):
  iden = iden_ref[...]
  for u in range(_RPK_U):
    pieces = [
        lax.dot_general(
            iden,
            t_ref[:, u * 512 + a * _LANES:u * 512 + (a + 1) * _LANES],
            (((1,), (1,)), ((), ())),
            preferred_element_type=jnp.float32)
        for a in range(4)
    ]
    out_ref[u * _LANES:(u + 1) * _LANES, :] = jnp.concatenate(pieces, axis=1)


def _repack_table(emb):
  v, d = emb.shape
  blk = 512 * _RPK_U
  tt = emb.T
  g = -(-v // blk)
  rows = blk * d // _LANES
  out = pl.pallas_call(
      _repack_body,
      grid=(g,),
      in_specs=[
          pl.BlockSpec((d, blk), lambda i: (0, i)),
          pl.BlockSpec((_LANES, _LANES), lambda i: (0, 0)),
      ],
      out_specs=pl.BlockSpec((rows, _LANES), lambda i: (i, 0)),
      out_shape=jax.ShapeDtypeStruct((g * rows, _LANES), jnp.float32),
  )(tt, jnp.eye(_LANES, dtype=jnp.float32))
  return out.reshape(g * blk, d)


def _remap_idx(v):
  return (v & ~511) + ((v & 127) << 2) + ((v >> 7) & 3)


def _mlp_body(p_ref, mm_ref, wp, wm, b1e, w2, b2, out_ref):
  h = (jnp.dot(p_ref[...], wp[...], preferred_element_type=jnp.float32)
       + jnp.dot(mm_ref[...], wm[...], preferred_element_type=jnp.float32)
       + b1e[...])
  h = jnp.maximum(h, 0.0)
  out_ref[...] = jnp.dot(h, w2[...], preferred_element_type=jnp.float32) + b2[...]


def _tc_mlp(p, mm2, wp, wm, b1e, w2, b2, bl=512):
  n, d_mm = mm2.shape
  d_dnn = wp.shape[1]
  d_hid = w2.shape[1]
  grid = (n // bl,)
  full = lambda r, c: pl.BlockSpec((r, c), lambda i: (0, 0))
  return pl.pallas_call(
      _mlp_body,
      grid=grid,
      in_specs=[
          pl.BlockSpec((bl, _LANES), lambda i: (i, 0)),
          pl.BlockSpec((bl, d_mm), lambda i: (i, 0)),
          full(_LANES, d_dnn),
          full(d_mm, d_dnn),
          full(1, d_dnn),
          full(d_dnn, d_hid),
          full(1, d_hid),
      ],
      out_specs=pl.BlockSpec((bl, d_hid), lambda i: (i, 0)),
      out_shape=jax.ShapeDtypeStruct((n, d_hid), jnp.float32),
  )(p, mm2, wp, wm, b1e, w2, b2)


def kernel(seq_id, item_mask, feat_cate, feat_brand, feat_price, feat_mm,
           emb_item, emb_cate, emb_brand, mm_W, mm_b, W1, b1, W2, b2):
  b, l = seq_id.shape
  n = b * l
  d_e = emb_item.shape[1]
  d_mm_in, d_mm = mm_W.shape
  d_dnn = W1.shape[1]
  assert n % (_NW * _CHUNK) == 0
  nch = n // (_NW * _CHUNK)

  ids = _remap_idx(
      (seq_id * item_mask).astype(jnp.int32)).T.reshape(_NW, nch, _CHUNK)
  idx_c = _remap_idx(
      feat_cate.astype(jnp.int32)).T.reshape(_NW, nch, _CHUNK)
  idx_b = _remap_idx(
      feat_brand.astype(jnp.int32)).T.reshape(_NW, nch, _CHUNK)
  price = feat_price.astype(jnp.float32).T.reshape(_NW, nch, _CHUNK)
  zeros = jnp.zeros((_GG * _CHUNK, d_e), jnp.float32)

  p = _sc_gather(ids, idx_c, idx_b, price, zeros,
                 _repack_table(emb_item), _repack_table(emb_cate),
                 _repack_table(emb_brand))

  n_emb = 3 * d_e + 1
  w1_mm = W1[n_emb:]
  wp = jnp.concatenate(
      [W1[:n_emb], jnp.zeros((_LANES - n_emb, d_dnn), W1.dtype)], axis=0)
  wm = mm_W @ w1_mm
  b1e = (b1 + mm_b @ w1_mm)[None, :]

  mm2 = jnp.transpose(feat_mm, (1, 0, 2)).reshape(n, d_mm_in)
  out = _tc_mlp(p, mm2, wp, wm, b1e, W2, b2[None, :])
  return jnp.transpose(out.reshape(l, b, out.shape[-1]), (1, 0, 2))


# speedup vs baseline: 1.4195x; 1.1254x over previous
import functools
import jax
import jax.numpy as jnp
from jax import lax
from jax.experimental import pallas as pl
from jax.experimental.pallas import tpu as pltpu
from jax.experimental.pallas import tpu_sc as plsc

_NC = 2
_NS = 16
_NW = _NC * _NS
_CHUNK = 128
_GG = 5
_LANES = 128


def _gather_body(nch, d_e, idx_i_hbm, idx_c_hbm, idx_b_hbm, price_hbm,
                 zeros_hbm, t_item, t_cate, t_brand, out_p,
                 idx_v, pr_v, rows_i, rows_c, rows_b, tail, gsem, wsem):
  wid = lax.axis_index("s") * _NC + lax.axis_index("c")
  base = wid * nch * _CHUNK
  rows_g = _GG * _CHUNK
  pltpu.sync_copy(idx_i_hbm.at[wid], idx_v.at[0])
  pltpu.sync_copy(idx_c_hbm.at[wid], idx_v.at[1])
  pltpu.sync_copy(idx_b_hbm.at[wid], idx_v.at[2])
  pltpu.sync_copy(price_hbm.at[wid], pr_v)
  pltpu.sync_copy(zeros_hbm, tail)

  tables = ((t_item, rows_i, 0), (t_cate, rows_c, 1), (t_brand, rows_b, 2))
  lane0 = jnp.zeros((16,), jnp.int32)
  iota16 = lax.iota(jnp.int32, 16)

  def group_body(g, _):
    row0 = base + g * rows_g
    cps = []
    for tbl, rows, t in tables:
      for j in range(_GG):
        cps.append(pltpu.async_copy(
            tbl.at[idx_v.at[t, g * _GG + j]],
            rows.at[pl.ds(j * _CHUNK, _CHUNK)], gsem))
    for j in range(_GG):
      c = g * _GG + j
      for k in range(_CHUNK // 16):
        pv = pr_v[c, pl.ds(16 * k, 16)]
        rows_idx = iota16 + (j * _CHUNK + 16 * k)
        plsc.store_scatter(tail, [rows_idx, lane0], pv)
    for cp in cps:
      cp.wait()
    wr = [pltpu.async_copy(
        rows, out_p.at[pl.ds(row0, rows_g), pl.ds(t * d_e, d_e)], wsem)
        for _tbl, rows, t in tables]
    wr.append(pltpu.async_copy(
        tail, out_p.at[pl.ds(row0, rows_g), pl.ds(3 * d_e, d_e)], wsem))
    for cp in wr:
      cp.wait()
    return 0

  lax.fori_loop(0, nch // _GG, group_body, 0)


def _sc_gather(idx_i, idx_c, idx_b, price, zeros, emb_item, emb_cate,
               emb_brand):
  nw, nch, chunk = idx_i.shape
  n = nw * nch * chunk
  d_e = emb_item.shape[1]
  mesh = plsc.VectorSubcoreMesh(core_axis_name="c", subcore_axis_name="s",
                                num_cores=_NC, num_subcores=_NS)
  body = functools.partial(_gather_body, nch, d_e)
  return pl.kernel(
      body,
      out_type=jax.ShapeDtypeStruct((n, _LANES), jnp.float32),
      mesh=mesh,
      scratch_types=[
          pltpu.VMEM((3, nch, chunk), jnp.int32),
          pltpu.VMEM((nch, chunk), jnp.float32),
          pltpu.VMEM((_GG * chunk, d_e), jnp.float32),
          pltpu.VMEM((_GG * chunk, d_e), jnp.float32),
          pltpu.VMEM((_GG * chunk, d_e), jnp.float32),
          pltpu.VMEM((_GG * chunk, d_e), jnp.float32),
          pltpu.SemaphoreType.DMA,
          pltpu.SemaphoreType.DMA,
      ],
      compiler_params=pltpu.CompilerParams(use_tc_tiling_on_sc=False,
                                           needs_layout_passes=False),
  )(idx_i, idx_c, idx_b, price, zeros, emb_item, emb_cate, emb_brand)


_RPK_U = 16


def _repack_body(t_ref, iden_ref, out_ref):
  iden = iden_ref[...]
  for u in range(_RPK_U):
    pieces = [
        lax.dot_general(
            iden,
            t_ref[:, u * 512 + a * _LANES:u * 512 + (a + 1) * _LANES],
            (((1,), (1,)), ((), ())),
            preferred_element_type=jnp.float32)
        for a in range(4)
    ]
    out_ref[u * _LANES:(u + 1) * _LANES, :] = jnp.concatenate(pieces, axis=1)


def _repack_table(emb):
  v, d = emb.shape
  blk = 512 * _RPK_U
  tt = emb.T
  g = -(-v // blk)
  rows = blk * d // _LANES
  out = pl.pallas_call(
      _repack_body,
      grid=(g,),
      in_specs=[
          pl.BlockSpec((d, blk), lambda i: (0, i)),
          pl.BlockSpec((_LANES, _LANES), lambda i: (0, 0)),
      ],
      out_specs=pl.BlockSpec((rows, _LANES), lambda i: (i, 0)),
      out_shape=jax.ShapeDtypeStruct((g * rows, _LANES), jnp.float32),
  )(tt, jnp.eye(_LANES, dtype=jnp.float32))
  return out.reshape(g * blk, d)


def _remap_idx(v):
  return (v & ~511) + ((v & 127) << 2) + ((v >> 7) & 3)


def _mlp_body(p_ref, mm_ref, wp, wm, b1e, w2, b2, out_ref):
  h = (jnp.dot(p_ref[...], wp[...], preferred_element_type=jnp.float32)
       + jnp.dot(mm_ref[...], wm[...], preferred_element_type=jnp.float32)
       + b1e[...])
  h = jnp.maximum(h, 0.0)
  out_ref[...] = jnp.dot(h, w2[...], preferred_element_type=jnp.float32) + b2[...]


def _tc_mlp(p, mm2, wp, wm, b1e, w2, b2, bl=1024):
  n, d_mm = mm2.shape
  d_dnn = wp.shape[1]
  d_hid = w2.shape[1]
  grid = (n // bl,)
  full = lambda r, c: pl.BlockSpec((r, c), lambda i: (0, 0))
  return pl.pallas_call(
      _mlp_body,
      grid=grid,
      in_specs=[
          pl.BlockSpec((bl, _LANES), lambda i: (i, 0)),
          pl.BlockSpec((bl, d_mm), lambda i: (i, 0)),
          full(_LANES, d_dnn),
          full(d_mm, d_dnn),
          full(1, d_dnn),
          full(d_dnn, d_hid),
          full(1, d_hid),
      ],
      out_specs=pl.BlockSpec((bl, d_hid), lambda i: (i, 0)),
      out_shape=jax.ShapeDtypeStruct((n, d_hid), jnp.float32),
  )(p, mm2, wp, wm, b1e, w2, b2)


def kernel(seq_id, item_mask, feat_cate, feat_brand, feat_price, feat_mm,
           emb_item, emb_cate, emb_brand, mm_W, mm_b, W1, b1, W2, b2):
  b, l = seq_id.shape
  n = b * l
  d_e = emb_item.shape[1]
  d_mm_in, d_mm = mm_W.shape
  d_dnn = W1.shape[1]
  assert n % (_NW * _CHUNK) == 0
  nch = n // (_NW * _CHUNK)

  ids = _remap_idx(
      (seq_id * item_mask).astype(jnp.int32)).T.reshape(_NW, nch, _CHUNK)
  idx_c = _remap_idx(
      feat_cate.astype(jnp.int32)).T.reshape(_NW, nch, _CHUNK)
  idx_b = _remap_idx(
      feat_brand.astype(jnp.int32)).T.reshape(_NW, nch, _CHUNK)
  price = feat_price.astype(jnp.float32).T.reshape(_NW, nch, _CHUNK)
  zeros = jnp.zeros((_GG * _CHUNK, d_e), jnp.float32)

  p = _sc_gather(ids, idx_c, idx_b, price, zeros,
                 _repack_table(emb_item), _repack_table(emb_cate),
                 _repack_table(emb_brand))

  n_emb = 3 * d_e + 1
  w1_mm = W1[n_emb:]
  wp = jnp.concatenate(
      [W1[:n_emb], jnp.zeros((_LANES - n_emb, d_dnn), W1.dtype)], axis=0)
  wm = mm_W @ w1_mm
  b1e = (b1 + mm_b @ w1_mm)[None, :]

  mm2 = jnp.transpose(feat_mm, (1, 0, 2)).reshape(n, d_mm_in)
  out = _tc_mlp(p, mm2, wp, wm, b1e, W2, b2[None, :])
  return jnp.transpose(out.reshape(l, b, out.shape[-1]), (1, 0, 2))
